# baseline (device time: 17992 ns/iter reference)
import jax
import jax.numpy as jnp
from jax import lax
from jax.experimental import pallas as pl
from jax.experimental.pallas import tpu as pltpu


NSPLIT = 2


def kernel(x):
    m, n = x.shape
    half = n // 2

    def body(x_ref, out_ref, send_sems, recv_sems):
        my_x = lax.axis_index("x")
        my_y = lax.axis_index("y")
        my_z = lax.axis_index("z")
        other_y = 1 - my_y
        peer = (my_x, other_y, my_z)

        barrier_sem = pltpu.get_barrier_semaphore()
        pl.semaphore_signal(
            barrier_sem, inc=1,
            device_id=peer, device_id_type=pl.DeviceIdType.MESH,
        )
        pl.semaphore_wait(barrier_sem, 1)

        rows = m // NSPLIT
        rdmas = []
        for s in range(NSPLIT):
            rdma = pltpu.make_async_remote_copy(
                src_ref=x_ref.at[pl.ds(s * rows, rows),
                                 pl.ds(other_y * half, half)],
                dst_ref=out_ref.at[pl.ds(my_y * m + s * rows, rows)],
                send_sem=send_sems.at[s],
                recv_sem=recv_sems.at[s],
                device_id=peer,
                device_id_type=pl.DeviceIdType.MESH,
            )
            rdma.start()
            rdmas.append(rdma)

        out_ref[pl.ds(my_y * m, m), :] = x_ref[:, pl.ds(my_y * half, half)]

        for rdma in rdmas:
            rdma.wait()

    return pl.pallas_call(
        body,
        out_shape=jax.ShapeDtypeStruct((2 * m, half), x.dtype),
        in_specs=[pl.BlockSpec(memory_space=pltpu.VMEM)],
        out_specs=pl.BlockSpec(memory_space=pltpu.VMEM),
        scratch_shapes=[
            pltpu.SemaphoreType.DMA((NSPLIT,)),
            pltpu.SemaphoreType.DMA((NSPLIT,)),
        ],
        compiler_params=pltpu.CompilerParams(collective_id=0),
    )(x)
